# baseline (speedup 1.0000x reference)
import functools

import jax
import jax.numpy as jnp
from jax import lax
from jax.experimental import pallas as pl
from jax.experimental.pallas import tpu as pltpu
from jax.experimental.pallas import tpu_sc as plsc

NC = 2
NS = 16
NW = NC * NS
L = 16

D = 64
DH = 32


def kernel(input, weight, fed_mask):
    B, S = input.shape
    V = weight.shape[0]
    n_rows = B * S
    per_w = n_rows // NW
    chunk = 1600
    n_chunks = per_w // chunk
    b_per_w = B // NW

    nblk = V // 128
    v_main = nblk * 128
    tail_n = V - v_main
    blk_w = 246
    start_last = nblk - blk_w
    ct_rows = V * DH // 128

    idxT = input.T.reshape(-1).astype(jnp.int32)
    wt = weight.T
    tail2 = weight[v_main:, :DH].reshape(tail_n * DH // 128, 128)

    mesh = plsc.VectorSubcoreMesh(
        core_axis_name="c", subcore_axis_name="s",
        num_cores=NC, num_subcores=NS)

    @functools.partial(
        pl.kernel,
        out_type=jax.ShapeDtypeStruct((ct_rows, 128), jnp.float32),
        mesh=mesh,
        compiler_params=pltpu.CompilerParams(needs_layout_passes=False),
        scratch_types=[
            pltpu.VMEM((DH, 128), jnp.float32),
            pltpu.VMEM((DH, 128), jnp.float32),
            pltpu.VMEM((DH, 128), jnp.float32),
            pltpu.VMEM((DH, 128), jnp.float32),
            pltpu.VMEM((tail_n * DH // 128, 128), jnp.float32),
            pltpu.SemaphoreType.DMA,
            pltpu.SemaphoreType.DMA,
            pltpu.SemaphoreType.DMA,
            pltpu.SemaphoreType.DMA,
        ],
    )
    def xpose(wt_hbm, tail_hbm, ct_hbm,
              tb0, tb1, ob0, ob1, ttbuf, is0, is1, os0, os1):
        wid = lax.axis_index("s") * NC + lax.axis_index("c")
        t0 = jnp.minimum(wid * (blk_w - 1), start_last)
        iota = lax.iota(jnp.int32, L)

        tbs = (tb0, tb1)
        obs = (ob0, ob1)
        iss = (is0, is1)
        oss = (os0, os1)

        def start(t, s):
            pltpu.make_async_copy(
                wt_hbm.at[pl.ds(0, DH), pl.ds(t * 128, 128)],
                tbs[s], iss[s]).start()

        def finish(t, s, not_first):
            pltpu.make_async_copy(
                wt_hbm.at[pl.ds(0, DH), pl.ds(t * 128, 128)],
                tbs[s], iss[s]).wait()

            @pl.when(not_first)
            def _():
                pltpu.make_async_copy(
                    obs[s], ct_hbm.at[pl.ds(0, DH)], oss[s]).wait()

            def tf(r, c2):
                for u in range(4):
                    vi = iota * 0 + (r * 4 + u)
                    a0 = plsc.load_gather(tbs[s], [iota, vi])
                    a1 = plsc.load_gather(tbs[s], [iota + L, vi])
                    obs[s][r, pl.ds(u * DH, L)] = a0
                    obs[s][r, pl.ds(u * DH + L, L)] = a1
                return c2
            lax.fori_loop(0, DH, tf, 0)
            pltpu.make_async_copy(
                obs[s], ct_hbm.at[pl.ds(t * DH, DH)], oss[s]).start()

        start(t0, 0)
        start(t0 + 1, 1)

        def pair(p, c):
            t = t0 + 2 * p
            finish(t, 0, p > 0)

            @pl.when(p < blk_w // 2 - 1)
            def _():
                start(t + 2, 0)
            finish(t + 1, 1, p > 0)

            @pl.when(p < blk_w // 2 - 1)
            def _():
                start(t + 3, 1)
            return c
        lax.fori_loop(0, blk_w // 2, pair, 0)

        pltpu.make_async_copy(ob0, ct_hbm.at[pl.ds(0, DH)], os0).wait()
        pltpu.make_async_copy(ob1, ct_hbm.at[pl.ds(0, DH)], os1).wait()

        @pl.when(wid == 0)
        def _():
            pltpu.sync_copy(tail_hbm, ttbuf)
            pltpu.sync_copy(
                ttbuf, ct_hbm.at[pl.ds(v_main * DH // 128,
                                       tail_n * DH // 128)])

    @functools.partial(
        pl.kernel,
        out_type=jax.ShapeDtypeStruct((n_rows, DH), jnp.float32),
        mesh=mesh,
        compiler_params=pltpu.CompilerParams(use_tc_tiling_on_sc=False),
        scratch_types=[
            pltpu.VMEM((chunk,), jnp.int32),
            pltpu.VMEM((chunk,), jnp.int32),
            pltpu.VMEM((chunk, DH), jnp.float32),
            pltpu.VMEM((chunk, DH), jnp.float32),
            pltpu.SemaphoreType.DMA,
            pltpu.SemaphoreType.DMA,
            pltpu.SemaphoreType.DMA,
            pltpu.SemaphoreType.DMA,
        ],
    )
    def rowgather(idx_hbm, ct_hbm, gi_hbm,
                  ix0, ix1, gb0, gb1, gs0, gs1, os0, os1):
        wid = lax.axis_index("s") * NC + lax.axis_index("c")
        base = wid * per_w
        ixs = (ix0, ix1)
        gbs = (gb0, gb1)
        gss = (gs0, gs1)
        oss = (os0, os1)

        def start(k, s):
            cb = base + k * chunk
            pltpu.sync_copy(idx_hbm.at[pl.ds(cb, chunk)], ixs[s])
            pltpu.make_async_copy(ct_hbm.at[ixs[s]], gbs[s], gss[s]).start()

        def finish(k, s, not_first):
            cb = base + k * chunk
            pltpu.make_async_copy(ct_hbm.at[ixs[s]], gbs[s], gss[s]).wait()

            @pl.when(not_first)
            def _():
                pltpu.make_async_copy(
                    gbs[s], gi_hbm.at[pl.ds(base, chunk)], oss[s]).wait()
            pltpu.make_async_copy(
                gbs[s], gi_hbm.at[pl.ds(cb, chunk)], oss[s]).start()

        start(0, 0)
        start(1, 1)
        for k in range(n_chunks):
            finish(k, k % 2, k >= 2)
            if k + 2 < n_chunks:
                start(k + 2, k % 2)
        pltpu.make_async_copy(
            gb0, gi_hbm.at[pl.ds(base, chunk)], os0).wait()
        pltpu.make_async_copy(
            gb1, gi_hbm.at[pl.ds(base, chunk)], os1).wait()

    @functools.partial(
        pl.kernel,
        out_type=jax.ShapeDtypeStruct((S, D, B), jnp.float32),
        mesh=mesh,
        compiler_params=pltpu.CompilerParams(needs_layout_passes=False),
        scratch_types=[
            pltpu.VMEM((b_per_w * DH,), jnp.float32),
            pltpu.VMEM((b_per_w * DH,), jnp.float32),
            pltpu.VMEM((D, 128), jnp.float32),
            pltpu.VMEM((D, 128), jnp.float32),
            pltpu.SemaphoreType.DMA,
            pltpu.SemaphoreType.DMA,
            pltpu.SemaphoreType.DMA,
            pltpu.SemaphoreType.DMA,
        ],
    )
    def xout(gi_hbm, oc_hbm, tb0, tb1, ob0, ob1, is0, is1, os0, os1):
        wid = lax.axis_index("s") * NC + lax.axis_index("c")
        b0 = wid * b_per_w
        iota = lax.iota(jnp.int32, L)
        z = jnp.zeros((L,), jnp.float32)

        tbs = (tb0, tb1)
        obs = (ob0, ob1)
        iss = (is0, is1)
        oss = (os0, os1)

        def zf(r, c):
            for g in range(8):
                ob0[r, pl.ds(g * L, L)] = z
                ob1[r, pl.ds(g * L, L)] = z
            return c
        lax.fori_loop(DH, D, zf, 0)

        def start(l, s):
            pltpu.make_async_copy(
                gi_hbm.at[pl.ds((l * B + b0) * DH, b_per_w * DH)],
                tbs[s], iss[s]).start()

        def finish(l, s, not_first):
            pltpu.make_async_copy(
                gi_hbm.at[pl.ds((l * B + b0) * DH, b_per_w * DH)],
                tbs[s], iss[s]).wait()

            @pl.when(not_first)
            def _():
                pltpu.make_async_copy(
                    obs[s], oc_hbm.at[0, :, pl.ds(b0, b_per_w)],
                    oss[s]).wait()

            for g in range(8):
                base_g = (iota + g * L) * DH
                for cc in range(DH):
                    v = plsc.load_gather(tbs[s], [base_g + cc])
                    obs[s][cc, pl.ds(g * L, L)] = v
            pltpu.make_async_copy(
                obs[s], oc_hbm.at[l, :, pl.ds(b0, b_per_w)], oss[s]).start()

        start(0, 0)

        def pair(p, c):
            l0 = 2 * p
            start(l0 + 1, 1)
            finish(l0, 0, p > 0)

            @pl.when(p < S // 2 - 1)
            def _():
                start(l0 + 2, 0)
            finish(l0 + 1, 1, p > 0)
            return c
        lax.fori_loop(0, S // 2, pair, 0)

        pltpu.make_async_copy(
            ob0, oc_hbm.at[0, :, pl.ds(b0, b_per_w)], os0).wait()
        pltpu.make_async_copy(
            ob1, oc_hbm.at[0, :, pl.ds(b0, b_per_w)], os1).wait()

    ct = xpose(wt, tail2)
    ct1 = ct.reshape(V, DH)
    gi = rowgather(idxT, ct1)
    gi1 = gi.reshape(-1)
    oc = xout(gi1)
    return oc.transpose(2, 0, 1)


# speedup vs baseline: 1.4810x; 1.4810x over previous
import functools

import jax
import jax.numpy as jnp
from jax import lax
from jax.experimental import pallas as pl
from jax.experimental.pallas import tpu as pltpu
from jax.experimental.pallas import tpu_sc as plsc

NC = 2
NS = 16
NW = NC * NS
L = 16

D = 64
DH = 32


def kernel(input, weight, fed_mask):
    B, S = input.shape
    V = weight.shape[0]
    n_rows = B * S
    per_w = n_rows // NW
    chunk = 1600
    n_chunks = per_w // chunk
    b_per_w = B // NW

    nblk = V // 128
    v_main = nblk * 128
    tail_n = V - v_main
    blk_w = 246
    start_last = nblk - blk_w
    ct_rows = V * DH // 128

    idxT = input.T.reshape(-1).astype(jnp.int32)
    wt = weight.T
    tail2 = weight[v_main:, :DH].reshape(tail_n * DH // 128, 128)

    mesh = plsc.VectorSubcoreMesh(
        core_axis_name="c", subcore_axis_name="s",
        num_cores=NC, num_subcores=NS)

    @functools.partial(
        pl.kernel,
        out_type=jax.ShapeDtypeStruct((ct_rows, 128), jnp.float32),
        mesh=mesh,
        compiler_params=pltpu.CompilerParams(needs_layout_passes=False),
        scratch_types=[
            pltpu.VMEM((DH, 128), jnp.float32),
            pltpu.VMEM((DH, 128), jnp.float32),
            pltpu.VMEM((DH, 128), jnp.float32),
            pltpu.VMEM((DH, 128), jnp.float32),
            pltpu.VMEM((tail_n * DH // 128, 128), jnp.float32),
            pltpu.SemaphoreType.DMA,
            pltpu.SemaphoreType.DMA,
            pltpu.SemaphoreType.DMA,
            pltpu.SemaphoreType.DMA,
        ],
    )
    def xpose(wt_hbm, tail_hbm, ct_hbm,
              tb0, tb1, ob0, ob1, ttbuf, is0, is1, os0, os1):
        wid = lax.axis_index("s") * NC + lax.axis_index("c")
        t0 = jnp.minimum(wid * (blk_w - 1), start_last)
        iota = lax.iota(jnp.int32, L)

        tbs = (tb0, tb1)
        obs = (ob0, ob1)
        iss = (is0, is1)
        oss = (os0, os1)

        def start(t, s):
            pltpu.make_async_copy(
                wt_hbm.at[pl.ds(0, DH), pl.ds(t * 128, 128)],
                tbs[s], iss[s]).start()

        def finish(t, s, not_first):
            pltpu.make_async_copy(
                wt_hbm.at[pl.ds(0, DH), pl.ds(t * 128, 128)],
                tbs[s], iss[s]).wait()

            @pl.when(not_first)
            def _():
                pltpu.make_async_copy(
                    obs[s], ct_hbm.at[pl.ds(0, DH)], oss[s]).wait()

            @plsc.parallel_loop(0, 128, unroll=8)
            def _(i):
                vi = iota * 0 + i
                a0 = plsc.load_gather(tbs[s], [iota, vi])
                a1 = plsc.load_gather(tbs[s], [iota + L, vi])
                r = i // 4
                off = pl.multiple_of((i % 4) * DH, DH)
                obs[s][r, pl.ds(off, L)] = a0
                obs[s][r, pl.ds(off + L, L)] = a1
            pltpu.make_async_copy(
                obs[s], ct_hbm.at[pl.ds(t * DH, DH)], oss[s]).start()

        start(t0, 0)
        start(t0 + 1, 1)

        def pair(p, c):
            t = t0 + 2 * p
            finish(t, 0, p > 0)

            @pl.when(p < blk_w // 2 - 1)
            def _():
                start(t + 2, 0)
            finish(t + 1, 1, p > 0)

            @pl.when(p < blk_w // 2 - 1)
            def _():
                start(t + 3, 1)
            return c
        lax.fori_loop(0, blk_w // 2, pair, 0)

        pltpu.make_async_copy(ob0, ct_hbm.at[pl.ds(0, DH)], os0).wait()
        pltpu.make_async_copy(ob1, ct_hbm.at[pl.ds(0, DH)], os1).wait()

        @pl.when(wid == 0)
        def _():
            pltpu.sync_copy(tail_hbm, ttbuf)
            pltpu.sync_copy(
                ttbuf, ct_hbm.at[pl.ds(v_main * DH // 128,
                                       tail_n * DH // 128)])

    @functools.partial(
        pl.kernel,
        out_type=jax.ShapeDtypeStruct((n_rows, DH), jnp.float32),
        mesh=mesh,
        compiler_params=pltpu.CompilerParams(use_tc_tiling_on_sc=False),
        scratch_types=[
            pltpu.VMEM((chunk,), jnp.int32),
            pltpu.VMEM((chunk,), jnp.int32),
            pltpu.VMEM((chunk, DH), jnp.float32),
            pltpu.VMEM((chunk, DH), jnp.float32),
            pltpu.SemaphoreType.DMA,
            pltpu.SemaphoreType.DMA,
            pltpu.SemaphoreType.DMA,
            pltpu.SemaphoreType.DMA,
        ],
    )
    def rowgather(idx_hbm, ct_hbm, gi_hbm,
                  ix0, ix1, gb0, gb1, gs0, gs1, os0, os1):
        wid = lax.axis_index("s") * NC + lax.axis_index("c")
        base = wid * per_w
        ixs = (ix0, ix1)
        gbs = (gb0, gb1)
        gss = (gs0, gs1)
        oss = (os0, os1)

        def start(k, s):
            cb = base + k * chunk
            pltpu.sync_copy(idx_hbm.at[pl.ds(cb, chunk)], ixs[s])
            pltpu.make_async_copy(ct_hbm.at[ixs[s]], gbs[s], gss[s]).start()

        def finish(k, s, not_first):
            cb = base + k * chunk
            pltpu.make_async_copy(ct_hbm.at[ixs[s]], gbs[s], gss[s]).wait()

            @pl.when(not_first)
            def _():
                pltpu.make_async_copy(
                    gbs[s], gi_hbm.at[pl.ds(base, chunk)], oss[s]).wait()
            pltpu.make_async_copy(
                gbs[s], gi_hbm.at[pl.ds(cb, chunk)], oss[s]).start()

        start(0, 0)
        start(1, 1)
        for k in range(n_chunks):
            finish(k, k % 2, k >= 2)
            if k + 2 < n_chunks:
                start(k + 2, k % 2)
        pltpu.make_async_copy(
            gb0, gi_hbm.at[pl.ds(base, chunk)], os0).wait()
        pltpu.make_async_copy(
            gb1, gi_hbm.at[pl.ds(base, chunk)], os1).wait()

    @functools.partial(
        pl.kernel,
        out_type=jax.ShapeDtypeStruct((S, D, B), jnp.float32),
        mesh=mesh,
        compiler_params=pltpu.CompilerParams(needs_layout_passes=False),
        scratch_types=[
            pltpu.VMEM((b_per_w * DH,), jnp.float32),
            pltpu.VMEM((b_per_w * DH,), jnp.float32),
            pltpu.VMEM((D, 128), jnp.float32),
            pltpu.VMEM((D, 128), jnp.float32),
            pltpu.SemaphoreType.DMA,
            pltpu.SemaphoreType.DMA,
            pltpu.SemaphoreType.DMA,
            pltpu.SemaphoreType.DMA,
        ],
    )
    def xout(gi_hbm, oc_hbm, tb0, tb1, ob0, ob1, is0, is1, os0, os1):
        wid = lax.axis_index("s") * NC + lax.axis_index("c")
        b0 = wid * b_per_w
        iota = lax.iota(jnp.int32, L)
        z = jnp.zeros((L,), jnp.float32)

        tbs = (tb0, tb1)
        obs = (ob0, ob1)
        iss = (is0, is1)
        oss = (os0, os1)

        def zf(r, c):
            for g in range(8):
                ob0[r, pl.ds(g * L, L)] = z
                ob1[r, pl.ds(g * L, L)] = z
            return c
        lax.fori_loop(DH, D, zf, 0)

        def start(l, s):
            pltpu.make_async_copy(
                gi_hbm.at[pl.ds((l * B + b0) * DH, b_per_w * DH)],
                tbs[s], iss[s]).start()

        def finish(l, s, not_first):
            pltpu.make_async_copy(
                gi_hbm.at[pl.ds((l * B + b0) * DH, b_per_w * DH)],
                tbs[s], iss[s]).wait()

            @pl.when(not_first)
            def _():
                pltpu.make_async_copy(
                    obs[s], oc_hbm.at[0, :, pl.ds(b0, b_per_w)],
                    oss[s]).wait()

            for g in range(8):
                base_g = (iota + g * L) * DH

                @plsc.parallel_loop(0, DH, unroll=8)
                def _(cc):
                    v = plsc.load_gather(tbs[s], [base_g + cc])
                    obs[s][cc, pl.ds(g * L, L)] = v
            pltpu.make_async_copy(
                obs[s], oc_hbm.at[l, :, pl.ds(b0, b_per_w)], oss[s]).start()

        start(0, 0)

        def pair(p, c):
            l0 = 2 * p
            start(l0 + 1, 1)
            finish(l0, 0, p > 0)

            @pl.when(p < S // 2 - 1)
            def _():
                start(l0 + 2, 0)
            finish(l0 + 1, 1, p > 0)
            return c
        lax.fori_loop(0, S // 2, pair, 0)

        pltpu.make_async_copy(
            ob0, oc_hbm.at[0, :, pl.ds(b0, b_per_w)], os0).wait()
        pltpu.make_async_copy(
            ob1, oc_hbm.at[0, :, pl.ds(b0, b_per_w)], os1).wait()

    ct = xpose(wt, tail2)
    ct1 = ct.reshape(V, DH)
    gi = rowgather(idxT, ct1)
    gi1 = gi.reshape(-1)
    oc = xout(gi1)
    return oc.transpose(2, 0, 1)


# speedup vs baseline: 1.4819x; 1.0006x over previous
import functools

import jax
import jax.numpy as jnp
from jax import lax
from jax.experimental import pallas as pl
from jax.experimental.pallas import tpu as pltpu
from jax.experimental.pallas import tpu_sc as plsc

NC = 2
NS = 16
NW = NC * NS
L = 16

D = 64
DH = 32


def kernel(input, weight, fed_mask):
    B, S = input.shape
    V = weight.shape[0]
    n_rows = B * S
    per_w = n_rows // NW
    chunk = 1600
    n_chunks = per_w // chunk
    b_per_w = B // NW

    nblk = V // 128
    v_main = nblk * 128
    tail_n = V - v_main
    blk_w = 246
    start_last = nblk - blk_w
    ct_rows = V * DH // 128

    idxT = input.T.reshape(-1).astype(jnp.int32)
    wt = weight.T
    tail2 = weight[v_main:, :DH].reshape(tail_n * DH // 128, 128)

    mesh = plsc.VectorSubcoreMesh(
        core_axis_name="c", subcore_axis_name="s",
        num_cores=NC, num_subcores=NS)

    @functools.partial(
        pl.kernel,
        out_type=jax.ShapeDtypeStruct((ct_rows, 128), jnp.float32),
        mesh=mesh,
        compiler_params=pltpu.CompilerParams(needs_layout_passes=False),
        scratch_types=[
            pltpu.VMEM((DH, 128), jnp.float32),
            pltpu.VMEM((DH, 128), jnp.float32),
            pltpu.VMEM((DH, 128), jnp.float32),
            pltpu.VMEM((DH, 128), jnp.float32),
            pltpu.VMEM((tail_n * DH // 128, 128), jnp.float32),
            pltpu.SemaphoreType.DMA,
            pltpu.SemaphoreType.DMA,
            pltpu.SemaphoreType.DMA,
            pltpu.SemaphoreType.DMA,
        ],
    )
    def xpose(wt_hbm, tail_hbm, ct_hbm,
              tb0, tb1, ob0, ob1, ttbuf, is0, is1, os0, os1):
        wid = lax.axis_index("s") * NC + lax.axis_index("c")
        t0 = jnp.minimum(wid * (blk_w - 1), start_last)
        iota = lax.iota(jnp.int32, L)

        tbs = (tb0, tb1)
        obs = (ob0, ob1)
        iss = (is0, is1)
        oss = (os0, os1)

        def start(t, s):
            pltpu.make_async_copy(
                wt_hbm.at[pl.ds(0, DH), pl.ds(t * 128, 128)],
                tbs[s], iss[s]).start()

        def finish(t, s, not_first):
            pltpu.make_async_copy(
                wt_hbm.at[pl.ds(0, DH), pl.ds(t * 128, 128)],
                tbs[s], iss[s]).wait()

            @pl.when(not_first)
            def _():
                pltpu.make_async_copy(
                    obs[s], ct_hbm.at[pl.ds(0, DH)], oss[s]).wait()

            @plsc.parallel_loop(0, 128, unroll=8)
            def _(i):
                vi = iota * 0 + i
                a0 = plsc.load_gather(tbs[s], [iota, vi])
                a1 = plsc.load_gather(tbs[s], [iota + L, vi])
                r = i // 4
                off = pl.multiple_of((i % 4) * DH, DH)
                obs[s][r, pl.ds(off, L)] = a0
                obs[s][r, pl.ds(off + L, L)] = a1
            pltpu.make_async_copy(
                obs[s], ct_hbm.at[pl.ds(t * DH, DH)], oss[s]).start()

        start(t0, 0)
        start(t0 + 1, 1)

        def pair(p, c):
            t = t0 + 2 * p
            finish(t, 0, p > 0)

            @pl.when(p < blk_w // 2 - 1)
            def _():
                start(t + 2, 0)
            finish(t + 1, 1, p > 0)

            @pl.when(p < blk_w // 2 - 1)
            def _():
                start(t + 3, 1)
            return c
        lax.fori_loop(0, blk_w // 2, pair, 0)

        pltpu.make_async_copy(ob0, ct_hbm.at[pl.ds(0, DH)], os0).wait()
        pltpu.make_async_copy(ob1, ct_hbm.at[pl.ds(0, DH)], os1).wait()

        @pl.when(wid == 0)
        def _():
            pltpu.sync_copy(tail_hbm, ttbuf)
            pltpu.sync_copy(
                ttbuf, ct_hbm.at[pl.ds(v_main * DH // 128,
                                       tail_n * DH // 128)])

    @functools.partial(
        pl.kernel,
        out_type=jax.ShapeDtypeStruct((n_rows, DH), jnp.float32),
        mesh=mesh,
        compiler_params=pltpu.CompilerParams(use_tc_tiling_on_sc=False),
        scratch_types=[
            pltpu.VMEM((chunk,), jnp.int32),
            pltpu.VMEM((chunk,), jnp.int32),
            pltpu.VMEM((chunk, DH), jnp.float32),
            pltpu.VMEM((chunk, DH), jnp.float32),
            pltpu.SemaphoreType.DMA,
            pltpu.SemaphoreType.DMA,
            pltpu.SemaphoreType.DMA,
            pltpu.SemaphoreType.DMA,
        ],
    )
    def rowgather(idx_hbm, ct_hbm, gi_hbm,
                  ix0, ix1, gb0, gb1, gs0, gs1, os0, os1):
        wid = lax.axis_index("s") * NC + lax.axis_index("c")
        base = wid * per_w
        ixs = (ix0, ix1)
        gbs = (gb0, gb1)
        gss = (gs0, gs1)
        oss = (os0, os1)

        def start(k, s, drain):
            cb = base + k * chunk
            pltpu.sync_copy(idx_hbm.at[pl.ds(cb, chunk)], ixs[s])
            if drain:
                pltpu.make_async_copy(
                    gbs[s], gi_hbm.at[pl.ds(base, chunk)], oss[s]).wait()
            pltpu.make_async_copy(ct_hbm.at[ixs[s]], gbs[s], gss[s]).start()

        def finish(k, s):
            cb = base + k * chunk
            pltpu.make_async_copy(ct_hbm.at[ixs[s]], gbs[s], gss[s]).wait()
            pltpu.make_async_copy(
                gbs[s], gi_hbm.at[pl.ds(cb, chunk)], oss[s]).start()

        start(0, 0, False)
        start(1, 1, False)
        for k in range(n_chunks):
            finish(k, k % 2)
            if k + 2 < n_chunks:
                start(k + 2, k % 2, True)
        pltpu.make_async_copy(
            gb0, gi_hbm.at[pl.ds(base, chunk)], os0).wait()
        pltpu.make_async_copy(
            gb1, gi_hbm.at[pl.ds(base, chunk)], os1).wait()

    @functools.partial(
        pl.kernel,
        out_type=jax.ShapeDtypeStruct((S, D, B), jnp.float32),
        mesh=mesh,
        compiler_params=pltpu.CompilerParams(needs_layout_passes=False),
        scratch_types=[
            pltpu.VMEM((b_per_w * DH,), jnp.float32),
            pltpu.VMEM((b_per_w * DH,), jnp.float32),
            pltpu.VMEM((D, 128), jnp.float32),
            pltpu.VMEM((D, 128), jnp.float32),
            pltpu.SemaphoreType.DMA,
            pltpu.SemaphoreType.DMA,
            pltpu.SemaphoreType.DMA,
            pltpu.SemaphoreType.DMA,
        ],
    )
    def xout(gi_hbm, oc_hbm, tb0, tb1, ob0, ob1, is0, is1, os0, os1):
        wid = lax.axis_index("s") * NC + lax.axis_index("c")
        b0 = wid * b_per_w
        iota = lax.iota(jnp.int32, L)
        z = jnp.zeros((L,), jnp.float32)

        tbs = (tb0, tb1)
        obs = (ob0, ob1)
        iss = (is0, is1)
        oss = (os0, os1)

        def zf(r, c):
            for g in range(8):
                ob0[r, pl.ds(g * L, L)] = z
                ob1[r, pl.ds(g * L, L)] = z
            return c
        lax.fori_loop(DH, D, zf, 0)

        def start(l, s):
            pltpu.make_async_copy(
                gi_hbm.at[pl.ds((l * B + b0) * DH, b_per_w * DH)],
                tbs[s], iss[s]).start()

        def finish(l, s, not_first):
            pltpu.make_async_copy(
                gi_hbm.at[pl.ds((l * B + b0) * DH, b_per_w * DH)],
                tbs[s], iss[s]).wait()

            @pl.when(not_first)
            def _():
                pltpu.make_async_copy(
                    obs[s], oc_hbm.at[0, :, pl.ds(b0, b_per_w)],
                    oss[s]).wait()

            for g in range(8):
                base_g = (iota + g * L) * DH

                @plsc.parallel_loop(0, DH, unroll=8)
                def _(cc):
                    v = plsc.load_gather(tbs[s], [base_g + cc])
                    obs[s][cc, pl.ds(g * L, L)] = v
            pltpu.make_async_copy(
                obs[s], oc_hbm.at[l, :, pl.ds(b0, b_per_w)], oss[s]).start()

        start(0, 0)

        def pair(p, c):
            l0 = 2 * p
            start(l0 + 1, 1)
            finish(l0, 0, p > 0)

            @pl.when(p < S // 2 - 1)
            def _():
                start(l0 + 2, 0)
            finish(l0 + 1, 1, p > 0)
            return c
        lax.fori_loop(0, S // 2, pair, 0)

        pltpu.make_async_copy(
            ob0, oc_hbm.at[0, :, pl.ds(b0, b_per_w)], os0).wait()
        pltpu.make_async_copy(
            ob1, oc_hbm.at[0, :, pl.ds(b0, b_per_w)], os1).wait()

    ct = xpose(wt, tail2)
    ct1 = ct.reshape(V, DH)
    gi = rowgather(idxT, ct1)
    gi1 = gi.reshape(-1)
    oc = xout(gi1)
    return oc.transpose(2, 0, 1)


# speedup vs baseline: 3.1144x; 2.1016x over previous
import functools

import jax
import jax.numpy as jnp
from jax import lax
from jax.experimental import pallas as pl
from jax.experimental.pallas import tpu as pltpu
from jax.experimental.pallas import tpu_sc as plsc

NC = 2
NS = 16
NW = NC * NS
L = 16

D = 64
DH = 32


def kernel(input, weight, fed_mask):
    B, S = input.shape
    V = weight.shape[0]
    n_rows = B * S
    per_w = n_rows // NW
    chunk = 1600
    n_chunks = per_w // chunk
    b_per_w = B // NW

    nblk = V // 128
    v_main = nblk * 128
    tail_n = V - v_main
    blk_w = 246
    start_last = nblk - blk_w
    ct_rows = V * DH // 128

    idxT = input.T.reshape(-1).astype(jnp.int32)
    wt = weight.T
    tail2 = weight[v_main:, :DH].reshape(tail_n * DH // 128, 128)

    mesh = plsc.VectorSubcoreMesh(
        core_axis_name="c", subcore_axis_name="s",
        num_cores=NC, num_subcores=NS)

    @functools.partial(
        pl.kernel,
        out_type=jax.ShapeDtypeStruct((ct_rows, 128), jnp.float32),
        mesh=mesh,
        compiler_params=pltpu.CompilerParams(needs_layout_passes=False),
        scratch_types=[
            pltpu.VMEM((DH, 128), jnp.float32),
            pltpu.VMEM((DH, 128), jnp.float32),
            pltpu.VMEM((DH, 128), jnp.float32),
            pltpu.VMEM((DH, 128), jnp.float32),
            pltpu.VMEM((tail_n * DH // 128, 128), jnp.float32),
            pltpu.SemaphoreType.DMA,
            pltpu.SemaphoreType.DMA,
            pltpu.SemaphoreType.DMA,
            pltpu.SemaphoreType.DMA,
        ],
    )
    def xpose(wt_hbm, tail_hbm, ct_hbm,
              tb0, tb1, ob0, ob1, ttbuf, is0, is1, os0, os1):
        wid = lax.axis_index("s") * NC + lax.axis_index("c")
        t0 = jnp.minimum(wid * (blk_w - 1), start_last)
        iota = lax.iota(jnp.int32, L)

        tbs = (tb0, tb1)
        obs = (ob0, ob1)
        iss = (is0, is1)
        oss = (os0, os1)

        def start(t, s):
            pltpu.make_async_copy(
                wt_hbm.at[pl.ds(0, DH), pl.ds(t * 128, 128)],
                tbs[s], iss[s]).start()

        def finish(t, s, not_first):
            pltpu.make_async_copy(
                wt_hbm.at[pl.ds(0, DH), pl.ds(t * 128, 128)],
                tbs[s], iss[s]).wait()

            @pl.when(not_first)
            def _():
                pltpu.make_async_copy(
                    obs[s], ct_hbm.at[pl.ds(0, DH)], oss[s]).wait()

            @plsc.parallel_loop(0, 128, unroll=8)
            def _(i):
                i2 = jnp.bitwise_and(iota + i, 127)
                a0 = plsc.load_gather(tbs[s], [iota, i2])
                a1 = plsc.load_gather(tbs[s], [iota + L, i2])
                r2 = lax.shift_right_logical(i2, 2)
                c2 = lax.shift_left(jnp.bitwise_and(i2, 3), 5) + iota
                plsc.store_scatter(obs[s], [r2, c2], a0)
                plsc.store_scatter(obs[s], [r2, c2 + L], a1)
            pltpu.make_async_copy(
                obs[s], ct_hbm.at[pl.ds(t * DH, DH)], oss[s]).start()

        start(t0, 0)
        start(t0 + 1, 1)

        def pair(p, c):
            t = t0 + 2 * p
            finish(t, 0, p > 0)

            @pl.when(p < blk_w // 2 - 1)
            def _():
                start(t + 2, 0)
            finish(t + 1, 1, p > 0)

            @pl.when(p < blk_w // 2 - 1)
            def _():
                start(t + 3, 1)
            return c
        lax.fori_loop(0, blk_w // 2, pair, 0)

        pltpu.make_async_copy(ob0, ct_hbm.at[pl.ds(0, DH)], os0).wait()
        pltpu.make_async_copy(ob1, ct_hbm.at[pl.ds(0, DH)], os1).wait()

        @pl.when(wid == 0)
        def _():
            pltpu.sync_copy(tail_hbm, ttbuf)
            pltpu.sync_copy(
                ttbuf, ct_hbm.at[pl.ds(v_main * DH // 128,
                                       tail_n * DH // 128)])

    @functools.partial(
        pl.kernel,
        out_type=jax.ShapeDtypeStruct((n_rows, DH), jnp.float32),
        mesh=mesh,
        compiler_params=pltpu.CompilerParams(use_tc_tiling_on_sc=False),
        scratch_types=[
            pltpu.VMEM((chunk,), jnp.int32),
            pltpu.VMEM((chunk,), jnp.int32),
            pltpu.VMEM((chunk, DH), jnp.float32),
            pltpu.VMEM((chunk, DH), jnp.float32),
            pltpu.SemaphoreType.DMA,
            pltpu.SemaphoreType.DMA,
            pltpu.SemaphoreType.DMA,
            pltpu.SemaphoreType.DMA,
        ],
    )
    def rowgather(idx_hbm, ct_hbm, gi_hbm,
                  ix0, ix1, gb0, gb1, gs0, gs1, os0, os1):
        wid = lax.axis_index("s") * NC + lax.axis_index("c")
        base = wid * per_w
        ixs = (ix0, ix1)
        gbs = (gb0, gb1)
        gss = (gs0, gs1)
        oss = (os0, os1)

        def start(k, s, drain):
            cb = base + k * chunk
            pltpu.sync_copy(idx_hbm.at[pl.ds(cb, chunk)], ixs[s])
            if drain:
                pltpu.make_async_copy(
                    gbs[s], gi_hbm.at[pl.ds(base, chunk)], oss[s]).wait()
            pltpu.make_async_copy(ct_hbm.at[ixs[s]], gbs[s], gss[s]).start()

        def finish(k, s):
            cb = base + k * chunk
            pltpu.make_async_copy(ct_hbm.at[ixs[s]], gbs[s], gss[s]).wait()
            pltpu.make_async_copy(
                gbs[s], gi_hbm.at[pl.ds(cb, chunk)], oss[s]).start()

        start(0, 0, False)
        start(1, 1, False)
        for k in range(n_chunks):
            finish(k, k % 2)
            if k + 2 < n_chunks:
                start(k + 2, k % 2, True)
        pltpu.make_async_copy(
            gb0, gi_hbm.at[pl.ds(base, chunk)], os0).wait()
        pltpu.make_async_copy(
            gb1, gi_hbm.at[pl.ds(base, chunk)], os1).wait()

    @functools.partial(
        pl.kernel,
        out_type=jax.ShapeDtypeStruct((S, D, B), jnp.float32),
        mesh=mesh,
        compiler_params=pltpu.CompilerParams(needs_layout_passes=False),
        scratch_types=[
            pltpu.VMEM((b_per_w * DH,), jnp.float32),
            pltpu.VMEM((b_per_w * DH,), jnp.float32),
            pltpu.VMEM((D, 128), jnp.float32),
            pltpu.VMEM((D, 128), jnp.float32),
            pltpu.SemaphoreType.DMA,
            pltpu.SemaphoreType.DMA,
            pltpu.SemaphoreType.DMA,
            pltpu.SemaphoreType.DMA,
        ],
    )
    def xout(gi_hbm, oc_hbm, tb0, tb1, ob0, ob1, is0, is1, os0, os1):
        wid = lax.axis_index("s") * NC + lax.axis_index("c")
        b0 = wid * b_per_w
        iota = lax.iota(jnp.int32, L)
        z = jnp.zeros((L,), jnp.float32)

        tbs = (tb0, tb1)
        obs = (ob0, ob1)
        iss = (is0, is1)
        oss = (os0, os1)

        def zf(r, c):
            for g in range(8):
                ob0[r, pl.ds(g * L, L)] = z
                ob1[r, pl.ds(g * L, L)] = z
            return c
        lax.fori_loop(DH, D, zf, 0)

        def start(l, s):
            pltpu.make_async_copy(
                gi_hbm.at[pl.ds((l * B + b0) * DH, b_per_w * DH)],
                tbs[s], iss[s]).start()

        def finish(l, s, not_first):
            pltpu.make_async_copy(
                gi_hbm.at[pl.ds((l * B + b0) * DH, b_per_w * DH)],
                tbs[s], iss[s]).wait()

            @pl.when(not_first)
            def _():
                pltpu.make_async_copy(
                    obs[s], oc_hbm.at[0, :, pl.ds(b0, b_per_w)],
                    oss[s]).wait()

            for g in range(8):
                base_g = (iota + g * L) * DH

                @plsc.parallel_loop(0, DH, unroll=8)
                def _(cc):
                    v = plsc.load_gather(tbs[s], [base_g + cc])
                    obs[s][cc, pl.ds(g * L, L)] = v
            pltpu.make_async_copy(
                obs[s], oc_hbm.at[l, :, pl.ds(b0, b_per_w)], oss[s]).start()

        start(0, 0)

        def pair(p, c):
            l0 = 2 * p
            start(l0 + 1, 1)
            finish(l0, 0, p > 0)

            @pl.when(p < S // 2 - 1)
            def _():
                start(l0 + 2, 0)
            finish(l0 + 1, 1, p > 0)
            return c
        lax.fori_loop(0, S // 2, pair, 0)

        pltpu.make_async_copy(
            ob0, oc_hbm.at[0, :, pl.ds(b0, b_per_w)], os0).wait()
        pltpu.make_async_copy(
            ob1, oc_hbm.at[0, :, pl.ds(b0, b_per_w)], os1).wait()

    ct = xpose(wt, tail2)
    ct1 = ct.reshape(V, DH)
    gi = rowgather(idxT, ct1)
    gi1 = gi.reshape(-1)
    oc = xout(gi1)
    return oc.transpose(2, 0, 1)


# speedup vs baseline: 3.7412x; 1.2013x over previous
import functools

import jax
import jax.numpy as jnp
from jax import lax
from jax.experimental import pallas as pl
from jax.experimental.pallas import tpu as pltpu
from jax.experimental.pallas import tpu_sc as plsc

NC = 2
NS = 16
NW = NC * NS
L = 16

D = 64
DH = 32


def kernel(input, weight, fed_mask):
    B, S = input.shape
    V = weight.shape[0]
    n_rows = B * S
    per_w = n_rows // NW
    chunk = 1600
    n_chunks = per_w // chunk
    b_per_w = B // NW

    nblk = V // 128
    v_main = nblk * 128
    tail_n = V - v_main
    blk_w = 246
    start_last = nblk - blk_w
    ct_rows = V * DH // 128

    idxT = input.T.reshape(-1).astype(jnp.int32)
    wt = weight.T
    tail2 = weight[v_main:, :DH].reshape(tail_n * DH // 128, 128)

    mesh = plsc.VectorSubcoreMesh(
        core_axis_name="c", subcore_axis_name="s",
        num_cores=NC, num_subcores=NS)

    @functools.partial(
        pl.kernel,
        out_type=jax.ShapeDtypeStruct((ct_rows, 128), jnp.float32),
        mesh=mesh,
        compiler_params=pltpu.CompilerParams(needs_layout_passes=False),
        scratch_types=[
            pltpu.VMEM((DH, 128), jnp.float32),
            pltpu.VMEM((DH, 128), jnp.float32),
            pltpu.VMEM((DH, 128), jnp.float32),
            pltpu.VMEM((DH, 128), jnp.float32),
            pltpu.VMEM((tail_n * DH // 128, 128), jnp.float32),
            pltpu.SemaphoreType.DMA,
            pltpu.SemaphoreType.DMA,
            pltpu.SemaphoreType.DMA,
            pltpu.SemaphoreType.DMA,
        ],
    )
    def xpose(wt_hbm, tail_hbm, ct_hbm,
              tb0, tb1, ob0, ob1, ttbuf, is0, is1, os0, os1):
        wid = lax.axis_index("s") * NC + lax.axis_index("c")
        t0 = jnp.minimum(wid * (blk_w - 1), start_last)
        iota = lax.iota(jnp.int32, L)

        tbs = (tb0, tb1)
        obs = (ob0, ob1)
        iss = (is0, is1)
        oss = (os0, os1)

        def start(t, s):
            pltpu.make_async_copy(
                wt_hbm.at[pl.ds(0, DH), pl.ds(t * 128, 128)],
                tbs[s], iss[s]).start()

        def finish(t, s, not_first):
            pltpu.make_async_copy(
                wt_hbm.at[pl.ds(0, DH), pl.ds(t * 128, 128)],
                tbs[s], iss[s]).wait()

            @pl.when(not_first)
            def _():
                pltpu.make_async_copy(
                    obs[s], ct_hbm.at[pl.ds(0, DH)], oss[s]).wait()

            @plsc.parallel_loop(0, 128, unroll=8)
            def _(i):
                i2 = jnp.bitwise_and(iota + i, 127)
                a0 = plsc.load_gather(tbs[s], [iota, i2])
                a1 = plsc.load_gather(tbs[s], [iota + L, i2])
                r2 = lax.shift_right_logical(i2, 2)
                c2 = lax.shift_left(jnp.bitwise_and(i2, 3), 5) + iota
                plsc.store_scatter(obs[s], [r2, c2], a0)
                plsc.store_scatter(obs[s], [r2, c2 + L], a1)
            pltpu.make_async_copy(
                obs[s], ct_hbm.at[pl.ds(t * DH, DH)], oss[s]).start()

        start(t0, 0)
        start(t0 + 1, 1)

        def pair(p, c):
            t = t0 + 2 * p
            finish(t, 0, p > 0)

            @pl.when(p < blk_w // 2 - 1)
            def _():
                start(t + 2, 0)
            finish(t + 1, 1, p > 0)

            @pl.when(p < blk_w // 2 - 1)
            def _():
                start(t + 3, 1)
            return c
        lax.fori_loop(0, blk_w // 2, pair, 0)

        pltpu.make_async_copy(ob0, ct_hbm.at[pl.ds(0, DH)], os0).wait()
        pltpu.make_async_copy(ob1, ct_hbm.at[pl.ds(0, DH)], os1).wait()

        @pl.when(wid == 0)
        def _():
            pltpu.sync_copy(tail_hbm, ttbuf)
            pltpu.sync_copy(
                ttbuf, ct_hbm.at[pl.ds(v_main * DH // 128,
                                       tail_n * DH // 128)])

    @functools.partial(
        pl.kernel,
        out_type=jax.ShapeDtypeStruct((n_rows, DH), jnp.float32),
        mesh=mesh,
        compiler_params=pltpu.CompilerParams(use_tc_tiling_on_sc=False),
        scratch_types=[
            pltpu.VMEM((chunk,), jnp.int32),
            pltpu.VMEM((chunk,), jnp.int32),
            pltpu.VMEM((chunk, DH), jnp.float32),
            pltpu.VMEM((chunk, DH), jnp.float32),
            pltpu.SemaphoreType.DMA,
            pltpu.SemaphoreType.DMA,
            pltpu.SemaphoreType.DMA,
            pltpu.SemaphoreType.DMA,
        ],
    )
    def rowgather(idx_hbm, ct_hbm, gi_hbm,
                  ix0, ix1, gb0, gb1, gs0, gs1, os0, os1):
        wid = lax.axis_index("s") * NC + lax.axis_index("c")
        base = wid * per_w
        ixs = (ix0, ix1)
        gbs = (gb0, gb1)
        gss = (gs0, gs1)
        oss = (os0, os1)

        def start(k, s, drain):
            cb = base + k * chunk
            pltpu.sync_copy(idx_hbm.at[pl.ds(cb, chunk)], ixs[s])
            if drain:
                pltpu.make_async_copy(
                    gbs[s], gi_hbm.at[pl.ds(base, chunk)], oss[s]).wait()
            pltpu.make_async_copy(ct_hbm.at[ixs[s]], gbs[s], gss[s]).start()

        def finish(k, s):
            cb = base + k * chunk
            pltpu.make_async_copy(ct_hbm.at[ixs[s]], gbs[s], gss[s]).wait()
            pltpu.make_async_copy(
                gbs[s], gi_hbm.at[pl.ds(cb, chunk)], oss[s]).start()

        start(0, 0, False)
        start(1, 1, False)
        for k in range(n_chunks):
            finish(k, k % 2)
            if k + 2 < n_chunks:
                start(k + 2, k % 2, True)
        pltpu.make_async_copy(
            gb0, gi_hbm.at[pl.ds(base, chunk)], os0).wait()
        pltpu.make_async_copy(
            gb1, gi_hbm.at[pl.ds(base, chunk)], os1).wait()

    @functools.partial(
        pl.kernel,
        out_type=jax.ShapeDtypeStruct((S, D, B), jnp.float32),
        mesh=mesh,
        compiler_params=pltpu.CompilerParams(needs_layout_passes=False),
        scratch_types=[
            pltpu.VMEM((b_per_w * DH,), jnp.float32),
            pltpu.VMEM((b_per_w * DH,), jnp.float32),
            pltpu.VMEM((D, 128), jnp.float32),
            pltpu.VMEM((D, 128), jnp.float32),
            pltpu.SemaphoreType.DMA,
            pltpu.SemaphoreType.DMA,
            pltpu.SemaphoreType.DMA,
            pltpu.SemaphoreType.DMA,
        ],
    )
    def xout(gi_hbm, oc_hbm, tb0, tb1, ob0, ob1, is0, is1, os0, os1):
        wid = lax.axis_index("s") * NC + lax.axis_index("c")
        b0 = wid * b_per_w
        iota = lax.iota(jnp.int32, L)
        z = jnp.zeros((L,), jnp.float32)

        tbs = (tb0, tb1)
        obs = (ob0, ob1)
        iss = (is0, is1)
        oss = (os0, os1)

        def zf(r, c):
            for g in range(8):
                ob0[r, pl.ds(g * L, L)] = z
                ob1[r, pl.ds(g * L, L)] = z
            return c
        lax.fori_loop(DH, D, zf, 0)

        def start(l, s):
            pltpu.make_async_copy(
                gi_hbm.at[pl.ds((l * B + b0) * DH, b_per_w * DH)],
                tbs[s], iss[s]).start()

        def finish(l, s, not_first):
            pltpu.make_async_copy(
                gi_hbm.at[pl.ds((l * B + b0) * DH, b_per_w * DH)],
                tbs[s], iss[s]).wait()

            @pl.when(not_first)
            def _():
                pltpu.make_async_copy(
                    obs[s], oc_hbm.at[0, :, pl.ds(b0, b_per_w)],
                    oss[s]).wait()

            for g in range(8):
                base_g = (iota + g * L) * DH
                colg = iota + g * L

                @plsc.parallel_loop(0, DH, unroll=8)
                def _(cc):
                    cc2 = jnp.bitwise_and(iota + cc, DH - 1)
                    v = plsc.load_gather(tbs[s], [base_g + cc2])
                    plsc.store_scatter(obs[s], [cc2, colg], v)
            pltpu.make_async_copy(
                obs[s], oc_hbm.at[l, :, pl.ds(b0, b_per_w)], oss[s]).start()

        start(0, 0)

        def pair(p, c):
            l0 = 2 * p
            start(l0 + 1, 1)
            finish(l0, 0, p > 0)

            @pl.when(p < S // 2 - 1)
            def _():
                start(l0 + 2, 0)
            finish(l0 + 1, 1, p > 0)
            return c
        lax.fori_loop(0, S // 2, pair, 0)

        pltpu.make_async_copy(
            ob0, oc_hbm.at[0, :, pl.ds(b0, b_per_w)], os0).wait()
        pltpu.make_async_copy(
            ob1, oc_hbm.at[0, :, pl.ds(b0, b_per_w)], os1).wait()

    ct = xpose(wt, tail2)
    ct1 = ct.reshape(V, DH)
    gi = rowgather(idxT, ct1)
    gi1 = gi.reshape(-1)
    oc = xout(gi1)
    return oc.transpose(2, 0, 1)


# speedup vs baseline: 4.4020x; 1.1766x over previous
import functools

import jax
import jax.numpy as jnp
from jax import lax
from jax.experimental import pallas as pl
from jax.experimental.pallas import tpu as pltpu
from jax.experimental.pallas import tpu_sc as plsc

NC = 2
NS = 16
NW = NC * NS
L = 16

D = 64
DH = 32


def kernel(input, weight, fed_mask):
    B, S = input.shape
    V = weight.shape[0]
    n_rows = B * S
    per_w = n_rows // NW
    chunk = 1600
    n_chunks = per_w // chunk
    b_per_w = B // NW

    nblk = V // 256
    v_main = nblk * 256
    tail_n = V - v_main
    blk_w = 124
    start_last = nblk - blk_w
    ct_rows = V * DH // 128

    idxT = input.T.reshape(-1).astype(jnp.int32)
    wt = weight.T
    tail2 = weight[v_main:, :DH].reshape(tail_n * DH // 128, 128)

    mesh = plsc.VectorSubcoreMesh(
        core_axis_name="c", subcore_axis_name="s",
        num_cores=NC, num_subcores=NS)

    @functools.partial(
        pl.kernel,
        out_type=jax.ShapeDtypeStruct((ct_rows, 128), jnp.float32),
        mesh=mesh,
        compiler_params=pltpu.CompilerParams(needs_layout_passes=False),
        scratch_types=[
            pltpu.VMEM((DH, 256), jnp.float32),
            pltpu.VMEM((DH, 256), jnp.float32),
            pltpu.VMEM((2 * DH, 128), jnp.float32),
            pltpu.VMEM((2 * DH, 128), jnp.float32),
            pltpu.VMEM((tail_n * DH // 128, 128), jnp.float32),
            pltpu.SemaphoreType.DMA,
            pltpu.SemaphoreType.DMA,
            pltpu.SemaphoreType.DMA,
            pltpu.SemaphoreType.DMA,
        ],
    )
    def xpose(wt_hbm, tail_hbm, ct_hbm,
              tb0, tb1, ob0, ob1, ttbuf, is0, is1, os0, os1):
        wid = lax.axis_index("s") * NC + lax.axis_index("c")
        t0 = jnp.minimum(wid * blk_w, start_last)
        iota = lax.iota(jnp.int32, L)

        tbs = (tb0, tb1)
        obs = (ob0, ob1)
        iss = (is0, is1)
        oss = (os0, os1)

        def start(t, s):
            pltpu.make_async_copy(
                wt_hbm.at[pl.ds(0, DH), pl.ds(t * 256, 256)],
                tbs[s], iss[s]).start()

        def finish(t, s, not_first):
            pltpu.make_async_copy(
                wt_hbm.at[pl.ds(0, DH), pl.ds(t * 256, 256)],
                tbs[s], iss[s]).wait()

            @pl.when(not_first)
            def _():
                pltpu.make_async_copy(
                    obs[s], ct_hbm.at[pl.ds(0, 2 * DH)], oss[s]).wait()

            @plsc.parallel_loop(0, 256, unroll=8)
            def _(i):
                i2 = jnp.bitwise_and(iota + i, 255)
                a0 = plsc.load_gather(tbs[s], [iota, i2])
                a1 = plsc.load_gather(tbs[s], [iota + L, i2])
                r2 = lax.shift_right_logical(i2, 2)
                c2 = lax.shift_left(jnp.bitwise_and(i2, 3), 5) + iota
                plsc.store_scatter(obs[s], [r2, c2], a0)
                plsc.store_scatter(obs[s], [r2, c2 + L], a1)
            pltpu.make_async_copy(
                obs[s], ct_hbm.at[pl.ds(t * 2 * DH, 2 * DH)], oss[s]).start()

        start(t0, 0)
        start(t0 + 1, 1)

        def pair(p, c):
            t = t0 + 2 * p
            finish(t, 0, p > 0)

            @pl.when(p < blk_w // 2 - 1)
            def _():
                start(t + 2, 0)
            finish(t + 1, 1, p > 0)

            @pl.when(p < blk_w // 2 - 1)
            def _():
                start(t + 3, 1)
            return c
        lax.fori_loop(0, blk_w // 2, pair, 0)

        pltpu.make_async_copy(ob0, ct_hbm.at[pl.ds(0, 2 * DH)], os0).wait()
        pltpu.make_async_copy(ob1, ct_hbm.at[pl.ds(0, 2 * DH)], os1).wait()

        @pl.when(wid == 0)
        def _():
            pltpu.sync_copy(tail_hbm, ttbuf)
            pltpu.sync_copy(
                ttbuf, ct_hbm.at[pl.ds(v_main * DH // 128,
                                       tail_n * DH // 128)])

    @functools.partial(
        pl.kernel,
        out_type=jax.ShapeDtypeStruct((n_rows, DH), jnp.float32),
        mesh=mesh,
        compiler_params=pltpu.CompilerParams(use_tc_tiling_on_sc=False),
        scratch_types=[
            pltpu.VMEM((chunk,), jnp.int32),
            pltpu.VMEM((chunk,), jnp.int32),
            pltpu.VMEM((chunk, DH), jnp.float32),
            pltpu.VMEM((chunk, DH), jnp.float32),
            pltpu.SemaphoreType.DMA,
            pltpu.SemaphoreType.DMA,
            pltpu.SemaphoreType.DMA,
            pltpu.SemaphoreType.DMA,
        ],
    )
    def rowgather(idx_hbm, ct_hbm, gi_hbm,
                  ix0, ix1, gb0, gb1, gs0, gs1, os0, os1):
        wid = lax.axis_index("s") * NC + lax.axis_index("c")
        base = wid * per_w
        ixs = (ix0, ix1)
        gbs = (gb0, gb1)
        gss = (gs0, gs1)
        oss = (os0, os1)

        def start(k, s, drain):
            cb = base + k * chunk
            pltpu.sync_copy(idx_hbm.at[pl.ds(cb, chunk)], ixs[s])
            if drain:
                pltpu.make_async_copy(
                    gbs[s], gi_hbm.at[pl.ds(base, chunk)], oss[s]).wait()
            pltpu.make_async_copy(ct_hbm.at[ixs[s]], gbs[s], gss[s]).start()

        def finish(k, s):
            cb = base + k * chunk
            pltpu.make_async_copy(ct_hbm.at[ixs[s]], gbs[s], gss[s]).wait()
            pltpu.make_async_copy(
                gbs[s], gi_hbm.at[pl.ds(cb, chunk)], oss[s]).start()

        start(0, 0, False)
        start(1, 1, False)
        for k in range(n_chunks):
            finish(k, k % 2)
            if k + 2 < n_chunks:
                start(k + 2, k % 2, True)
        pltpu.make_async_copy(
            gb0, gi_hbm.at[pl.ds(base, chunk)], os0).wait()
        pltpu.make_async_copy(
            gb1, gi_hbm.at[pl.ds(base, chunk)], os1).wait()

    @functools.partial(
        pl.kernel,
        out_type=jax.ShapeDtypeStruct((S, D, B), jnp.float32),
        mesh=mesh,
        compiler_params=pltpu.CompilerParams(needs_layout_passes=False),
        scratch_types=[
            pltpu.VMEM((b_per_w * DH,), jnp.float32),
            pltpu.VMEM((b_per_w * DH,), jnp.float32),
            pltpu.VMEM((D, 128), jnp.float32),
            pltpu.VMEM((D, 128), jnp.float32),
            pltpu.SemaphoreType.DMA,
            pltpu.SemaphoreType.DMA,
            pltpu.SemaphoreType.DMA,
            pltpu.SemaphoreType.DMA,
        ],
    )
    def xout(gi_hbm, oc_hbm, tb0, tb1, ob0, ob1, is0, is1, os0, os1):
        wid = lax.axis_index("s") * NC + lax.axis_index("c")
        b0 = wid * b_per_w
        iota = lax.iota(jnp.int32, L)
        z = jnp.zeros((L,), jnp.float32)

        tbs = (tb0, tb1)
        obs = (ob0, ob1)
        iss = (is0, is1)
        oss = (os0, os1)

        def zf(r, c):
            for g in range(8):
                ob0[r, pl.ds(g * L, L)] = z
                ob1[r, pl.ds(g * L, L)] = z
            return c
        lax.fori_loop(DH, D, zf, 0)

        def start(l, s):
            pltpu.make_async_copy(
                gi_hbm.at[pl.ds((l * B + b0) * DH, b_per_w * DH)],
                tbs[s], iss[s]).start()

        def finish(l, s, not_first):
            pltpu.make_async_copy(
                gi_hbm.at[pl.ds((l * B + b0) * DH, b_per_w * DH)],
                tbs[s], iss[s]).wait()

            @pl.when(not_first)
            def _():
                pltpu.make_async_copy(
                    obs[s], oc_hbm.at[0, :, pl.ds(b0, b_per_w)],
                    oss[s]).wait()

            for g in range(8):
                base_g = (iota + g * L) * DH
                colg = iota + g * L

                @plsc.parallel_loop(0, DH, unroll=8)
                def _(cc):
                    cc2 = jnp.bitwise_and(iota + cc, DH - 1)
                    v = plsc.load_gather(tbs[s], [base_g + cc2])
                    plsc.store_scatter(obs[s], [cc2, colg], v)
            pltpu.make_async_copy(
                obs[s], oc_hbm.at[l, :, pl.ds(b0, b_per_w)], oss[s]).start()

        start(0, 0)

        def pair(p, c):
            l0 = 2 * p
            start(l0 + 1, 1)
            finish(l0, 0, p > 0)

            @pl.when(p < S // 2 - 1)
            def _():
                start(l0 + 2, 0)
            finish(l0 + 1, 1, p > 0)
            return c
        lax.fori_loop(0, S // 2, pair, 0)

        pltpu.make_async_copy(
            ob0, oc_hbm.at[0, :, pl.ds(b0, b_per_w)], os0).wait()
        pltpu.make_async_copy(
            ob1, oc_hbm.at[0, :, pl.ds(b0, b_per_w)], os1).wait()

    ct = xpose(wt, tail2)
    ct1 = ct.reshape(V, DH)
    gi = rowgather(idxT, ct1)
    gi1 = gi.reshape(-1)
    oc = xout(gi1)
    return oc.transpose(2, 0, 1)
